# Initial kernel scaffold; baseline (speedup 1.0000x reference)
import functools

import jax
import jax.numpy as jnp
from jax import lax
from jax.experimental import pallas as pl
from jax.experimental.pallas import tpu as pltpu
from jax.experimental.pallas import tpu_sc as plsc

_D = 32
_IB = 128
_TOTAL = 16384 * 50
_NROW = _TOTAL // _IB
_NC = 2
_NS = 16
_NW = _NC * _NS
_RPW = _NROW // _NW
_K = 8
_STEPS = _RPW // _K

_mesh = plsc.VectorSubcoreMesh(core_axis_name="c", subcore_axis_name="s")


@functools.partial(
    pl.kernel,
    out_type=jax.ShapeDtypeStruct((_NROW, _IB, _D), jnp.float32),
    mesh=_mesh,
    scratch_types=[
        pltpu.VMEM((_RPW, _IB), jnp.int32),
        pltpu.VMEM((_K, _IB, _D), jnp.float32),
        pltpu.SemaphoreType.DMA,
    ],
)
def _lookup(data_hbm, idx_hbm, out_hbm, idx_v, rows_v, sem):
    wid = lax.axis_index("s") * _NC + lax.axis_index("c")
    base = wid * _RPW
    pltpu.sync_copy(idx_hbm.at[pl.ds(base, _RPW)], idx_v)

    def body(j, _):
        pltpu.async_copy(
            data_hbm.at[idx_v.at[pl.ds(j * _K, _K)]], rows_v, sem
        ).wait()
        pltpu.sync_copy(rows_v, out_hbm.at[pl.ds(base + j * _K, _K)])
        return ()

    lax.fori_loop(0, _STEPS, body, (), unroll=False)


def kernel(data, input_ids):
    ids = input_ids.astype(jnp.int32).reshape(_NROW, _IB)
    out = _lookup(data, ids)
    return out.reshape(input_ids.shape[0], input_ids.shape[1], _D)


# speedup vs baseline: 1.6122x; 1.6122x over previous
import functools

import jax
import jax.numpy as jnp
from jax import lax
from jax.experimental import pallas as pl
from jax.experimental.pallas import tpu as pltpu
from jax.experimental.pallas import tpu_sc as plsc

_D = 32
_B = 16384
_S = 50
_NC = 2
_NS = 16
_NW = _NC * _NS
_CPW = _B // _NW
_NBUF = 4
_LA = 2

_mesh = plsc.VectorSubcoreMesh(core_axis_name="c", subcore_axis_name="s")


@functools.partial(
    pl.kernel,
    out_type=jax.ShapeDtypeStruct((_B, _S, _D), jnp.float32),
    mesh=_mesh,
    scratch_types=[
        pltpu.VMEM((_CPW, _S), jnp.int32),
        pltpu.VMEM((_NBUF, _S, _D), jnp.float32),
        pltpu.SemaphoreType.DMA((_NBUF,)),
        pltpu.SemaphoreType.DMA((_NBUF,)),
    ],
    compiler_params=pltpu.CompilerParams(use_tc_tiling_on_sc=False),
)
def _lookup(data_hbm, idx_hbm, out_hbm, idx_v, rows_v, gsem, osem):
    wid = lax.axis_index("s") * _NC + lax.axis_index("c")
    base = wid * _CPW
    pltpu.sync_copy(idx_hbm.at[pl.ds(base, _CPW)], idx_v)

    for b in range(_LA):
        pltpu.async_copy(data_hbm.at[idx_v.at[b]], rows_v.at[b], gsem.at[b])

    def group(g, _):
        for b in range(_NBUF):
            j = g * _NBUF + b
            b2 = (b + _LA) % _NBUF
            pltpu.make_async_copy(
                data_hbm.at[idx_v.at[j]], rows_v.at[b], gsem.at[b]
            ).wait()
            pltpu.make_async_copy(
                rows_v.at[b], out_hbm.at[base + j], osem.at[b]
            ).start()
            nj = j + _LA

            @pl.when(nj >= _NBUF)
            def _():
                pltpu.make_async_copy(
                    rows_v.at[b2],
                    out_hbm.at[base + nj - _NBUF],
                    osem.at[b2],
                ).wait()

            @pl.when(nj < _CPW)
            def _():
                pltpu.async_copy(
                    data_hbm.at[idx_v.at[nj]], rows_v.at[b2], gsem.at[b2]
                )
        return ()

    lax.fori_loop(0, _CPW // _NBUF, group, (), unroll=False)
    for k in range(_CPW - (_NBUF - _LA), _CPW):
        b = k % _NBUF
        pltpu.make_async_copy(
            rows_v.at[b], out_hbm.at[base + k], osem.at[b]
        ).wait()


def kernel(data, input_ids):
    return _lookup(data, input_ids.astype(jnp.int32))


# speedup vs baseline: 1.7834x; 1.1062x over previous
import functools

import jax
import jax.numpy as jnp
from jax import lax
from jax.experimental import pallas as pl
from jax.experimental.pallas import tpu as pltpu
from jax.experimental.pallas import tpu_sc as plsc

_D = 32
_B = 16384
_S = 50
_NC = 2
_NS = 16
_NW = _NC * _NS
_CPW = _B // _NW
_NBUF = 8
_LA = 6

_mesh = plsc.VectorSubcoreMesh(core_axis_name="c", subcore_axis_name="s")


@functools.partial(
    pl.kernel,
    out_type=jax.ShapeDtypeStruct((_B, _S, _D), jnp.float32),
    mesh=_mesh,
    scratch_types=[
        pltpu.VMEM((_CPW, _S), jnp.int32),
        pltpu.VMEM((_NBUF, _S, _D), jnp.float32),
        pltpu.SemaphoreType.DMA((_NBUF,)),
        pltpu.SemaphoreType.DMA((_NBUF,)),
    ],
    compiler_params=pltpu.CompilerParams(use_tc_tiling_on_sc=False),
)
def _lookup(data_hbm, idx_hbm, out_hbm, idx_v, rows_v, gsem, osem):
    wid = lax.axis_index("s") * _NC + lax.axis_index("c")
    base = wid * _CPW
    pltpu.sync_copy(idx_hbm.at[pl.ds(base, _CPW)], idx_v)

    for b in range(_LA):
        pltpu.async_copy(data_hbm.at[idx_v.at[b]], rows_v.at[b], gsem.at[b])

    def group(g, _):
        for b in range(_NBUF):
            j = g * _NBUF + b
            b2 = (b + _LA) % _NBUF
            pltpu.make_async_copy(
                data_hbm.at[idx_v.at[j]], rows_v.at[b], gsem.at[b]
            ).wait()
            pltpu.make_async_copy(
                rows_v.at[b], out_hbm.at[base + j], osem.at[b]
            ).start()
            nj = j + _LA

            @pl.when(nj >= _NBUF)
            def _():
                pltpu.make_async_copy(
                    rows_v.at[b2],
                    out_hbm.at[base + nj - _NBUF],
                    osem.at[b2],
                ).wait()

            @pl.when(nj < _CPW)
            def _():
                pltpu.async_copy(
                    data_hbm.at[idx_v.at[nj]], rows_v.at[b2], gsem.at[b2]
                )
        return ()

    lax.fori_loop(0, _CPW // _NBUF, group, (), unroll=False)
    for k in range(_CPW - (_NBUF - _LA), _CPW):
        b = k % _NBUF
        pltpu.make_async_copy(
            rows_v.at[b], out_hbm.at[base + k], osem.at[b]
        ).wait()


def kernel(data, input_ids):
    return _lookup(data, input_ids.astype(jnp.int32))
